# Initial kernel scaffold; baseline (speedup 1.0000x reference)
import jax
import jax.numpy as jnp
from jax.experimental import pallas as pl


def kernel(heads, rels, tails, years, months, days, neighbor_idx, neighbor_rel, ny, nm, nd, ent_embs, rel_embs, y_freq, y_phi, y_amp, m_freq, m_phi, m_amp, d_freq, d_phi, d_amp, W, b):
    raise NotImplementedError("write your pallas kernel here")


# speedup vs baseline: 4.9407x; 4.9407x over previous
import functools

import jax
import jax.numpy as jnp
from jax import lax
from jax.experimental import pallas as pl
from jax.experimental.pallas import tpu as pltpu
from jax.experimental.pallas import tpu_sc as plsc

B = 1024
K = 20
NUM_ENT = 10000
NUM_REL = 16
S = 96
T = 32
D = S + T
R2 = 2 * NUM_REL
NTOK = 2 * B * K
BLK = 1280
EPB = BLK // K
NBLK = NTOK // BLK
TBW = S + 9 * T

NW = 32
TPW = NTOK // NW
CH = 256
NCH = TPW // CH


@functools.cache
def _build_sc_gather():
    @functools.partial(
        pl.kernel,
        out_type=jax.ShapeDtypeStruct((NTOK, TBW), jnp.float32),
        mesh=plsc.VectorSubcoreMesh(core_axis_name="c", subcore_axis_name="s"),
        scratch_types=[
            pltpu.VMEM((TPW,), jnp.int32),
            pltpu.VMEM((CH, TBW), jnp.float32),
            pltpu.SemaphoreType.DMA,
        ],
    )
    def _sc_gather(table_hbm, idx_hbm, out_hbm, idx_v, rows_v, sem):
        wid = lax.axis_index("s") * 2 + lax.axis_index("c")
        base = wid * TPW
        pltpu.sync_copy(idx_hbm.at[pl.ds(base, TPW)], idx_v)
        for c in range(NCH):
            pltpu.async_copy(
                table_hbm.at[idx_v.at[pl.ds(c * CH, CH)]], rows_v, sem
            ).wait()
            pltpu.sync_copy(rows_v, out_hbm.at[pl.ds(base + c * CH, CH)])

    return _sc_gather


def _transform_body(g_ref, tv_ref, rel_ref, w_ref, b_ref, out_ref):
    g = g_ref[...]
    e = g[:, 0:S]
    ty = tv_ref[:, 0:1]
    tm = tv_ref[:, 1:2]
    td = tv_ref[:, 2:3]
    yf = g[:, 96:128]
    yp = g[:, 128:160]
    ya = g[:, 160:192]
    mf = g[:, 192:224]
    mp = g[:, 224:256]
    ma = g[:, 256:288]
    df = g[:, 288:320]
    dp = g[:, 320:352]
    da = g[:, 352:384]
    temb = (ya * jnp.sin(yf * ty + yp)
            + ma * jnp.sin(mf * tm + mp)
            + da * jnp.sin(df * td + dp))
    x = jnp.concatenate([e, temb], axis=1)

    rel = rel_ref[...]
    ri = lax.broadcasted_iota(jnp.int32, (BLK, R2), 1)
    oh = (rel == ri).astype(jnp.float32)
    w = w_ref[...]
    acc = lax.dot_general(oh, b_ref[...], (((1,), (0,)), ((), ())),
                          preferred_element_type=jnp.float32)
    for r in range(R2):
        zr = lax.dot_general(x, w[r], (((1,), (0,)), ((), ())),
                             preferred_element_type=jnp.float32)
        acc = acc + oh[:, r:r + 1] * zr
    acc = jnp.maximum(acc, 0.0)

    prow = lax.broadcasted_iota(jnp.int32, (EPB, BLK), 0)
    pcol = lax.broadcasted_iota(jnp.int32, (EPB, BLK), 1)
    pool = jnp.where(pcol // K == prow, 1.0 / K, 0.0)
    out_ref[...] = lax.dot_general(pool, acc, (((1,), (0,)), ((), ())),
                                   preferred_element_type=jnp.float32)


_transform = pl.pallas_call(
    _transform_body,
    grid=(NBLK,),
    in_specs=[
        pl.BlockSpec((BLK, TBW), lambda i: (i, 0)),
        pl.BlockSpec((BLK, 3), lambda i: (i, 0)),
        pl.BlockSpec((BLK, 1), lambda i: (i, 0)),
        pl.BlockSpec((R2, D, D), lambda i: (0, 0, 0)),
        pl.BlockSpec((R2, D), lambda i: (0, 0)),
    ],
    out_specs=pl.BlockSpec((EPB, D), lambda i: (i, 0)),
    out_shape=jax.ShapeDtypeStruct((2 * B, D), jnp.float32),
)


def _score_body(p_ref, r_ref, re_ref, o_ref):
    h = p_ref[0:B, :]
    t = p_ref[B:2 * B, :]
    ridx = r_ref[...]
    i16 = lax.broadcasted_iota(jnp.int32, (B, NUM_REL), 1)
    oh = (ridx == i16).astype(jnp.float32)
    rr = lax.dot_general(oh, re_ref[...], (((1,), (0,)), ((), ())),
                         preferred_element_type=jnp.float32)
    diff = h + rr - t
    s = jnp.sum(diff * diff, axis=1, keepdims=True)
    o_ref[...] = -jnp.sqrt(s + 1e-12)


_score = pl.pallas_call(
    _score_body,
    in_specs=[
        pl.BlockSpec((2 * B, D), lambda: (0, 0)),
        pl.BlockSpec((B, 1), lambda: (0, 0)),
        pl.BlockSpec((NUM_REL, D), lambda: (0, 0)),
    ],
    out_specs=pl.BlockSpec((B, 1), lambda: (0, 0)),
    out_shape=jax.ShapeDtypeStruct((B, 1), jnp.float32),
)


def kernel(heads, rels, tails, years, months, days, neighbor_idx, neighbor_rel,
           ny, nm, nd, ent_embs, rel_embs,
           y_freq, y_phi, y_amp, m_freq, m_phi, m_amp,
           d_freq, d_phi, d_amp, W, b):
    table = jnp.concatenate(
        [ent_embs, y_freq, y_phi, y_amp, m_freq, m_phi, m_amp,
         d_freq, d_phi, d_amp], axis=1)
    idx = neighbor_idx.reshape(NTOK).astype(jnp.int32)
    g = _build_sc_gather()(table, idx)
    tv = jnp.concatenate([ny, nm, nd], axis=-1).reshape(NTOK, 3)
    rel = neighbor_rel.reshape(NTOK, 1).astype(jnp.int32)
    pooled = _transform(g, tv, rel, W, b)
    scores = _score(pooled, rels.reshape(B, 1).astype(jnp.int32), rel_embs)
    return scores.reshape(B)


# speedup vs baseline: 5.0185x; 1.0158x over previous
import functools

import jax
import jax.numpy as jnp
from jax import lax
from jax.experimental import pallas as pl
from jax.experimental.pallas import tpu as pltpu
from jax.experimental.pallas import tpu_sc as plsc

B = 1024
K = 20
NUM_ENT = 10000
NUM_REL = 16
S = 96
T = 32
D = S + T
R2 = 2 * NUM_REL
NTOK = 2 * B * K
BLK = 1280
EPB = BLK // K
NBLK = NTOK // BLK
TBW = S + 9 * T

NW = 32
TPW = NTOK // NW
CH = 256
NCH = TPW // CH


@functools.cache
def _build_sc_gather():
    @functools.partial(
        pl.kernel,
        out_type=jax.ShapeDtypeStruct((NTOK, TBW), jnp.float32),
        mesh=plsc.VectorSubcoreMesh(core_axis_name="c", subcore_axis_name="s"),
        scratch_types=[
            pltpu.VMEM((TPW,), jnp.int32),
            pltpu.VMEM((CH, TBW), jnp.float32),
            pltpu.SemaphoreType.DMA,
        ],
    )
    def _sc_gather(table_hbm, idx_hbm, out_hbm, idx_v, rows_v, sem):
        wid = lax.axis_index("s") * 2 + lax.axis_index("c")
        base = wid * TPW
        pltpu.sync_copy(idx_hbm.at[pl.ds(base, TPW)], idx_v)
        for c in range(NCH):
            pltpu.async_copy(
                table_hbm.at[idx_v.at[pl.ds(c * CH, CH)]], rows_v, sem
            ).wait()
            pltpu.sync_copy(rows_v, out_hbm.at[pl.ds(base + c * CH, CH)])

    return _sc_gather


def _transform_body(g_ref, tv_ref, rel_ref, w_ref, b_ref, out_ref):
    g = g_ref[...]
    e = g[:, 0:S]
    ty = tv_ref[:, 0:1]
    tm = tv_ref[:, 1:2]
    td = tv_ref[:, 2:3]
    yf = g[:, 96:128]
    yp = g[:, 128:160]
    ya = g[:, 160:192]
    mf = g[:, 192:224]
    mp = g[:, 224:256]
    ma = g[:, 256:288]
    df = g[:, 288:320]
    dp = g[:, 320:352]
    da = g[:, 352:384]
    temb = (ya * jnp.sin(yf * ty + yp)
            + ma * jnp.sin(mf * tm + mp)
            + da * jnp.sin(df * td + dp))
    x = jnp.concatenate([e, temb], axis=1)

    rel = rel_ref[...]
    ri = lax.broadcasted_iota(jnp.int32, (BLK, R2), 1)
    oh = (rel == ri).astype(jnp.float32)
    w = w_ref[...]
    xb = x.astype(jnp.bfloat16)
    acc = lax.dot_general(oh, b_ref[...], (((1,), (0,)), ((), ())),
                          preferred_element_type=jnp.float32)
    for r in range(R2):
        zr = lax.dot_general(xb, w[r], (((1,), (0,)), ((), ())),
                             preferred_element_type=jnp.float32)
        acc = acc + oh[:, r:r + 1] * zr
    acc = jnp.maximum(acc, 0.0)

    prow = lax.broadcasted_iota(jnp.int32, (EPB, BLK), 0)
    pcol = lax.broadcasted_iota(jnp.int32, (EPB, BLK), 1)
    pool = jnp.where(pcol // K == prow, 1.0 / K, 0.0)
    out_ref[...] = lax.dot_general(pool, acc, (((1,), (0,)), ((), ())),
                                   preferred_element_type=jnp.float32)


_transform = pl.pallas_call(
    _transform_body,
    grid=(NBLK,),
    in_specs=[
        pl.BlockSpec((BLK, TBW), lambda i: (i, 0)),
        pl.BlockSpec((BLK, 3), lambda i: (i, 0)),
        pl.BlockSpec((BLK, 1), lambda i: (i, 0)),
        pl.BlockSpec((R2, D, D), lambda i: (0, 0, 0)),
        pl.BlockSpec((R2, D), lambda i: (0, 0)),
    ],
    out_specs=pl.BlockSpec((EPB, D), lambda i: (i, 0)),
    out_shape=jax.ShapeDtypeStruct((2 * B, D), jnp.float32),
)


def _score_body(p_ref, r_ref, re_ref, o_ref):
    h = p_ref[0:B, :]
    t = p_ref[B:2 * B, :]
    ridx = r_ref[...]
    i16 = lax.broadcasted_iota(jnp.int32, (B, NUM_REL), 1)
    oh = (ridx == i16).astype(jnp.float32)
    rr = lax.dot_general(oh, re_ref[...], (((1,), (0,)), ((), ())),
                         preferred_element_type=jnp.float32)
    diff = h + rr - t
    s = jnp.sum(diff * diff, axis=1, keepdims=True)
    o_ref[...] = -jnp.sqrt(s + 1e-12)


_score = pl.pallas_call(
    _score_body,
    in_specs=[
        pl.BlockSpec((2 * B, D), lambda: (0, 0)),
        pl.BlockSpec((B, 1), lambda: (0, 0)),
        pl.BlockSpec((NUM_REL, D), lambda: (0, 0)),
    ],
    out_specs=pl.BlockSpec((B, 1), lambda: (0, 0)),
    out_shape=jax.ShapeDtypeStruct((B, 1), jnp.float32),
)


def kernel(heads, rels, tails, years, months, days, neighbor_idx, neighbor_rel,
           ny, nm, nd, ent_embs, rel_embs,
           y_freq, y_phi, y_amp, m_freq, m_phi, m_amp,
           d_freq, d_phi, d_amp, W, b):
    table = jnp.concatenate(
        [ent_embs, y_freq, y_phi, y_amp, m_freq, m_phi, m_amp,
         d_freq, d_phi, d_amp], axis=1)
    idx = neighbor_idx.reshape(NTOK).astype(jnp.int32)
    g = _build_sc_gather()(table, idx)
    tv = jnp.concatenate([ny, nm, nd], axis=-1).reshape(NTOK, 3)
    rel = neighbor_rel.reshape(NTOK, 1).astype(jnp.int32)
    pooled = _transform(g, tv, rel, W.astype(jnp.bfloat16), b)
    scores = _score(pooled, rels.reshape(B, 1).astype(jnp.int32), rel_embs)
    return scores.reshape(B)


# speedup vs baseline: 8.0950x; 1.6130x over previous
import functools

import jax
import jax.numpy as jnp
from jax import lax
from jax.experimental import pallas as pl
from jax.experimental.pallas import tpu as pltpu
from jax.experimental.pallas import tpu_sc as plsc

B = 1024
K = 20
NUM_ENT = 10000
NUM_REL = 16
S = 96
T = 32
D = S + T
R2 = 2 * NUM_REL
NTOK = 2 * B * K
BLK = 1280
EPB = BLK // K
NBLK = NTOK // BLK
TBW = 256

NW = 32
TPW = NTOK // NW
CH = 256
NCH = TPW // CH


@functools.cache
def _build_sc_gather():
    @functools.partial(
        pl.kernel,
        out_type=jax.ShapeDtypeStruct((NTOK, TBW), jnp.int32),
        mesh=plsc.VectorSubcoreMesh(core_axis_name="c", subcore_axis_name="s"),
        scratch_types=[
            pltpu.VMEM((TPW,), jnp.int32),
            pltpu.VMEM((CH, TBW), jnp.int32),
            pltpu.SemaphoreType.DMA,
        ],
    )
    def _sc_gather(table_hbm, idx_hbm, out_hbm, idx_v, rows_v, sem):
        wid = lax.axis_index("s") * 2 + lax.axis_index("c")
        base = wid * TPW
        pltpu.sync_copy(idx_hbm.at[pl.ds(base, TPW)], idx_v)
        for c in range(NCH):
            pltpu.async_copy(
                table_hbm.at[idx_v.at[pl.ds(c * CH, CH)]], rows_v, sem
            ).wait()
            pltpu.sync_copy(rows_v, out_hbm.at[pl.ds(base + c * CH, CH)])

    return _sc_gather


def _transform_body(g_ref, tv_ref, rel_ref, w_ref, b_ref, out_ref):
    gx = g_ref[:, 0:128]
    gy = g_ref[:, 128:256]
    hmask = jnp.int32(-65536)
    g0 = lax.bitcast_convert_type(gx << 16, jnp.float32)
    fr = lax.bitcast_convert_type(gx & hmask, jnp.float32)
    ph = lax.bitcast_convert_type(gy << 16, jnp.float32)
    am = lax.bitcast_convert_type(gy & hmask, jnp.float32)

    sl = lax.broadcasted_iota(jnp.int32, (4, 128), 1)
    sc = lax.broadcasted_iota(jnp.int32, (4, 128), 0)
    sel = jnp.where((sl < S) & (sl // T == sc), 1.0, 0.0)
    tval = lax.dot_general(tv_ref[...], sel, (((1,), (0,)), ((), ())),
                           preferred_element_type=jnp.float32)

    sv = (am * jnp.sin(fr * tval + ph)).astype(jnp.bfloat16)
    fl = lax.broadcasted_iota(jnp.int32, (128, 128), 0)
    fo = lax.broadcasted_iota(jnp.int32, (128, 128), 1)
    fold = jnp.where((fl < S) & (fo == S + fl % T), 1.0, 0.0).astype(jnp.bfloat16)
    temb = lax.dot_general(sv, fold, (((1,), (0,)), ((), ())),
                           preferred_element_type=jnp.float32)
    xb = (g0 + temb).astype(jnp.bfloat16)

    rel = rel_ref[...]
    ri = lax.broadcasted_iota(jnp.int32, (BLK, R2), 1)
    oh = (rel == ri).astype(jnp.float32)
    acc = lax.dot_general(oh, b_ref[...], (((1,), (0,)), ((), ())),
                          preferred_element_type=jnp.float32)
    for r in range(R2):
        zr = lax.dot_general(xb, w_ref[r], (((1,), (0,)), ((), ())),
                             preferred_element_type=jnp.float32)
        acc = acc + oh[:, r:r + 1] * zr
    acc = jnp.maximum(acc, 0.0)

    prow = lax.broadcasted_iota(jnp.int32, (EPB, BLK), 0)
    pcol = lax.broadcasted_iota(jnp.int32, (EPB, BLK), 1)
    pool = jnp.where(pcol // K == prow, 1.0 / K, 0.0)
    out_ref[...] = lax.dot_general(pool, acc, (((1,), (0,)), ((), ())),
                                   preferred_element_type=jnp.float32)


_transform = pl.pallas_call(
    _transform_body,
    grid=(NBLK,),
    in_specs=[
        pl.BlockSpec((BLK, TBW), lambda i: (i, 0)),
        pl.BlockSpec((BLK, 4), lambda i: (i, 0)),
        pl.BlockSpec((BLK, 1), lambda i: (i, 0)),
        pl.BlockSpec((R2, D, D), lambda i: (0, 0, 0)),
        pl.BlockSpec((R2, D), lambda i: (0, 0)),
    ],
    out_specs=pl.BlockSpec((EPB, D), lambda i: (i, 0)),
    out_shape=jax.ShapeDtypeStruct((2 * B, D), jnp.float32),
)


def _score_body(p_ref, r_ref, re_ref, o_ref):
    h = p_ref[0:B, :]
    t = p_ref[B:2 * B, :]
    ridx = r_ref[...]
    i16 = lax.broadcasted_iota(jnp.int32, (B, NUM_REL), 1)
    oh = (ridx == i16).astype(jnp.float32)
    rr = lax.dot_general(oh, re_ref[...], (((1,), (0,)), ((), ())),
                         preferred_element_type=jnp.float32)
    diff = h + rr - t
    s = jnp.sum(diff * diff, axis=1, keepdims=True)
    o_ref[...] = -jnp.sqrt(s + 1e-12)


_score = pl.pallas_call(
    _score_body,
    in_specs=[
        pl.BlockSpec((2 * B, D), lambda: (0, 0)),
        pl.BlockSpec((B, 1), lambda: (0, 0)),
        pl.BlockSpec((NUM_REL, D), lambda: (0, 0)),
    ],
    out_specs=pl.BlockSpec((B, 1), lambda: (0, 0)),
    out_shape=jax.ShapeDtypeStruct((B, 1), jnp.float32),
)


def kernel(heads, rels, tails, years, months, days, neighbor_idx, neighbor_rel,
           ny, nm, nd, ent_embs, rel_embs,
           y_freq, y_phi, y_amp, m_freq, m_phi, m_amp,
           d_freq, d_phi, d_amp, W, b):
    zpad = jnp.zeros((NUM_ENT, T), jnp.float32)
    ent_g = jnp.concatenate([ent_embs, zpad], axis=1)
    fr_g = jnp.concatenate([y_freq, m_freq, d_freq, zpad], axis=1)
    ph_g = jnp.concatenate([y_phi, m_phi, d_phi, zpad], axis=1)
    am_g = jnp.concatenate([y_amp, m_amp, d_amp, zpad], axis=1)

    def pack2(lo, hi):
        lob = lax.bitcast_convert_type(lo.astype(jnp.bfloat16),
                                       jnp.uint16).astype(jnp.uint32)
        hib = lax.bitcast_convert_type(hi.astype(jnp.bfloat16),
                                       jnp.uint16).astype(jnp.uint32)
        return lax.bitcast_convert_type(lob | (hib << 16), jnp.int32)

    table = jnp.concatenate([pack2(ent_g, fr_g), pack2(ph_g, am_g)], axis=1)
    idx = neighbor_idx.reshape(NTOK).astype(jnp.int32)
    g = _build_sc_gather()(table, idx)
    tv = jnp.concatenate(
        [ny, nm, nd, jnp.zeros_like(ny)], axis=-1).reshape(NTOK, 4)
    rel = neighbor_rel.reshape(NTOK, 1).astype(jnp.int32)
    pooled = _transform(g, tv, rel, W.astype(jnp.bfloat16), b)
    scores = _score(pooled, rels.reshape(B, 1).astype(jnp.int32), rel_embs)
    return scores.reshape(B)


# speedup vs baseline: 9.5421x; 1.1788x over previous
import functools

import jax
import jax.numpy as jnp
from jax import lax
from jax.experimental import pallas as pl
from jax.experimental.pallas import tpu as pltpu
from jax.experimental.pallas import tpu_sc as plsc

B = 1024
K = 20
NUM_ENT = 10000
NUM_REL = 16
S = 96
T = 32
D = S + T
R2 = 2 * NUM_REL
NTOK = 2 * B * K
BLK = 1280
EPB = BLK // K
NBLK = NTOK // BLK
TBW = 256

NW = 32
TPW = NTOK // NW
CH = 256
NCH = TPW // CH


@functools.cache
def _build_sc_gather():
    @functools.partial(
        pl.kernel,
        out_type=jax.ShapeDtypeStruct((NTOK, TBW), jnp.int32),
        mesh=plsc.VectorSubcoreMesh(core_axis_name="c", subcore_axis_name="s"),
        scratch_types=[
            pltpu.VMEM((TPW,), jnp.int32),
            pltpu.VMEM((CH, TBW), jnp.int32),
            pltpu.SemaphoreType.DMA,
        ],
    )
    def _sc_gather(table_hbm, idx_hbm, out_hbm, idx_v, rows_v, sem):
        wid = lax.axis_index("s") * 2 + lax.axis_index("c")
        base = wid * TPW
        pltpu.sync_copy(idx_hbm.at[pl.ds(base, TPW)], idx_v)
        for c in range(NCH):
            pltpu.async_copy(
                table_hbm.at[idx_v.at[pl.ds(c * CH, CH)]], rows_v, sem
            ).wait()
            pltpu.sync_copy(rows_v, out_hbm.at[pl.ds(base + c * CH, CH)])

    return _sc_gather


def _transform_body(g_ref, tv_ref, rel_ref, w_ref, b_ref, out_ref):
    gx = g_ref[:, 0:128]
    gy = g_ref[:, 128:256]
    hmask = jnp.int32(-65536)
    g0 = lax.bitcast_convert_type(gx << 16, jnp.float32)
    fr = lax.bitcast_convert_type(gx & hmask, jnp.float32)
    ph = lax.bitcast_convert_type(gy << 16, jnp.float32)
    am = lax.bitcast_convert_type(gy & hmask, jnp.float32)

    sl = lax.broadcasted_iota(jnp.int32, (4, 128), 1)
    sc = lax.broadcasted_iota(jnp.int32, (4, 128), 0)
    sel = jnp.where((sl < S) & (sl // T == sc), 1.0, 0.0)
    tval = lax.dot_general(tv_ref[...], sel, (((1,), (0,)), ((), ())),
                           preferred_element_type=jnp.float32)

    a = fr * tval + ph
    magic = jnp.float32(12582912.0)
    nf = a * jnp.float32(0.3183098861837907) + magic
    nint = lax.bitcast_convert_type(nf, jnp.int32) - jnp.int32(0x4B400000)
    ni = nint.astype(jnp.float32)
    sgn = 1.0 - 2.0 * (nint & 1).astype(jnp.float32)
    r = a - ni * jnp.float32(3.140625)
    r = r - ni * jnp.float32(0.0009676536)
    r = r - ni * jnp.float32(5.126688e-12)
    r2 = r * r
    p = r * (jnp.float32(9.999999970017e-01)
             + r2 * (jnp.float32(-1.666665997157e-01)
                     + r2 * (jnp.float32(8.333097587152e-03)
                             + r2 * (jnp.float32(-1.981248784256e-04)
                                     + r2 * jnp.float32(2.612907779947e-06)))))
    sv = (am * (p * sgn)).astype(jnp.bfloat16)
    fl = lax.broadcasted_iota(jnp.int32, (128, 128), 0)
    fo = lax.broadcasted_iota(jnp.int32, (128, 128), 1)
    fold = jnp.where((fl < S) & (fo == S + fl % T), 1.0, 0.0).astype(jnp.bfloat16)
    temb = lax.dot_general(sv, fold, (((1,), (0,)), ((), ())),
                           preferred_element_type=jnp.float32)
    xb = (g0 + temb).astype(jnp.bfloat16)

    rel = rel_ref[...]
    ri = lax.broadcasted_iota(jnp.int32, (BLK, R2), 1)
    oh = (rel == ri).astype(jnp.float32)
    acc = lax.dot_general(oh, b_ref[...], (((1,), (0,)), ((), ())),
                          preferred_element_type=jnp.float32)
    for r in range(R2):
        zr = lax.dot_general(xb, w_ref[r], (((1,), (0,)), ((), ())),
                             preferred_element_type=jnp.float32)
        acc = acc + oh[:, r:r + 1] * zr
    acc = jnp.maximum(acc, 0.0)

    prow = lax.broadcasted_iota(jnp.int32, (EPB, BLK), 0)
    pcol = lax.broadcasted_iota(jnp.int32, (EPB, BLK), 1)
    pool = jnp.where(pcol // K == prow, 1.0 / K, 0.0)
    out_ref[...] = lax.dot_general(pool, acc, (((1,), (0,)), ((), ())),
                                   preferred_element_type=jnp.float32)


_transform = pl.pallas_call(
    _transform_body,
    grid=(NBLK,),
    in_specs=[
        pl.BlockSpec((BLK, TBW), lambda i: (i, 0)),
        pl.BlockSpec((BLK, 4), lambda i: (i, 0)),
        pl.BlockSpec((BLK, 1), lambda i: (i, 0)),
        pl.BlockSpec((R2, D, D), lambda i: (0, 0, 0)),
        pl.BlockSpec((R2, D), lambda i: (0, 0)),
    ],
    out_specs=pl.BlockSpec((EPB, D), lambda i: (i, 0)),
    out_shape=jax.ShapeDtypeStruct((2 * B, D), jnp.float32),
)


def _score_body(p_ref, r_ref, re_ref, o_ref):
    h = p_ref[0:B, :]
    t = p_ref[B:2 * B, :]
    ridx = r_ref[...]
    i16 = lax.broadcasted_iota(jnp.int32, (B, NUM_REL), 1)
    oh = (ridx == i16).astype(jnp.float32)
    rr = lax.dot_general(oh, re_ref[...], (((1,), (0,)), ((), ())),
                         preferred_element_type=jnp.float32)
    diff = h + rr - t
    s = jnp.sum(diff * diff, axis=1, keepdims=True)
    o_ref[...] = -jnp.sqrt(s + 1e-12)


_score = pl.pallas_call(
    _score_body,
    in_specs=[
        pl.BlockSpec((2 * B, D), lambda: (0, 0)),
        pl.BlockSpec((B, 1), lambda: (0, 0)),
        pl.BlockSpec((NUM_REL, D), lambda: (0, 0)),
    ],
    out_specs=pl.BlockSpec((B, 1), lambda: (0, 0)),
    out_shape=jax.ShapeDtypeStruct((B, 1), jnp.float32),
)


def kernel(heads, rels, tails, years, months, days, neighbor_idx, neighbor_rel,
           ny, nm, nd, ent_embs, rel_embs,
           y_freq, y_phi, y_amp, m_freq, m_phi, m_amp,
           d_freq, d_phi, d_amp, W, b):
    zpad = jnp.zeros((NUM_ENT, T), jnp.float32)
    ent_g = jnp.concatenate([ent_embs, zpad], axis=1)
    fr_g = jnp.concatenate([y_freq, m_freq, d_freq, zpad], axis=1)
    ph_g = jnp.concatenate([y_phi, m_phi, d_phi, zpad], axis=1)
    am_g = jnp.concatenate([y_amp, m_amp, d_amp, zpad], axis=1)

    def pack2(lo, hi):
        lob = lax.bitcast_convert_type(lo.astype(jnp.bfloat16),
                                       jnp.uint16).astype(jnp.uint32)
        hib = lax.bitcast_convert_type(hi.astype(jnp.bfloat16),
                                       jnp.uint16).astype(jnp.uint32)
        return lax.bitcast_convert_type(lob | (hib << 16), jnp.int32)

    table = jnp.concatenate([pack2(ent_g, fr_g), pack2(ph_g, am_g)], axis=1)
    idx = neighbor_idx.reshape(NTOK).astype(jnp.int32)
    g = _build_sc_gather()(table, idx)
    tv = jnp.concatenate(
        [ny, nm, nd, jnp.zeros_like(ny)], axis=-1).reshape(NTOK, 4)
    rel = neighbor_rel.reshape(NTOK, 1).astype(jnp.int32)
    pooled = _transform(g, tv, rel, W.astype(jnp.bfloat16), b)
    scores = _score(pooled, rels.reshape(B, 1).astype(jnp.int32), rel_embs)
    return scores.reshape(B)


# speedup vs baseline: 9.6068x; 1.0068x over previous
import functools

import jax
import jax.numpy as jnp
from jax import lax
from jax.experimental import pallas as pl
from jax.experimental.pallas import tpu as pltpu
from jax.experimental.pallas import tpu_sc as plsc

B = 1024
K = 20
NUM_ENT = 10000
NUM_REL = 16
S = 96
T = 32
D = S + T
R2 = 2 * NUM_REL
NTOK = 2 * B * K
BLK = 1280
EPB = BLK // K
NBLK = NTOK // BLK
TBW = 256

NW = 32
TPW = NTOK // NW
CH = 160
NCH = TPW // CH


@functools.cache
def _build_sc_gather():
    @functools.partial(
        pl.kernel,
        out_type=jax.ShapeDtypeStruct((NTOK, TBW), jnp.int32),
        mesh=plsc.VectorSubcoreMesh(core_axis_name="c", subcore_axis_name="s"),
        scratch_types=[
            pltpu.VMEM((TPW,), jnp.int32),
            pltpu.VMEM((CH, TBW), jnp.int32),
            pltpu.VMEM((CH, TBW), jnp.int32),
            pltpu.SemaphoreType.DMA,
            pltpu.SemaphoreType.DMA,
        ],
    )
    def _sc_gather(table_hbm, idx_hbm, out_hbm, idx_v, rows_a, rows_b, s_a, s_b):
        wid = lax.axis_index("s") * 2 + lax.axis_index("c")
        base = wid * TPW
        pltpu.sync_copy(idx_hbm.at[pl.ds(base, TPW)], idx_v)
        bufs = (rows_a, rows_b)
        sems = (s_a, s_b)
        pending = [None, None]
        pending[0] = pltpu.async_copy(
            table_hbm.at[idx_v.at[pl.ds(0, CH)]], bufs[0], sems[0])
        for c in range(NCH):
            if c + 1 < NCH:
                nxt = (c + 1) % 2
                pending[nxt] = pltpu.async_copy(
                    table_hbm.at[idx_v.at[pl.ds((c + 1) * CH, CH)]],
                    bufs[nxt], sems[nxt])
            cur = c % 2
            pending[cur].wait()
            pltpu.sync_copy(bufs[cur], out_hbm.at[pl.ds(base + c * CH, CH)])

    return _sc_gather


def _transform_body(g_ref, tv_ref, rel_ref, w_ref, b_ref, rq_ref, re_ref,
                    out_ref, pool_ref):
    gx = g_ref[:, 0:128]
    gy = g_ref[:, 128:256]
    hmask = jnp.int32(-65536)
    g0 = lax.bitcast_convert_type(gx << 16, jnp.float32)
    fr = lax.bitcast_convert_type(gx & hmask, jnp.float32)
    ph = lax.bitcast_convert_type(gy << 16, jnp.float32)
    am = lax.bitcast_convert_type(gy & hmask, jnp.float32)

    sl = lax.broadcasted_iota(jnp.int32, (4, 128), 1)
    sc = lax.broadcasted_iota(jnp.int32, (4, 128), 0)
    sel = jnp.where((sl < S) & (sl // T == sc), 1.0, 0.0)
    tval = lax.dot_general(tv_ref[...], sel, (((1,), (0,)), ((), ())),
                           preferred_element_type=jnp.float32)

    a = fr * tval + ph
    magic = jnp.float32(12582912.0)
    nf = a * jnp.float32(0.3183098861837907) + magic
    nint = lax.bitcast_convert_type(nf, jnp.int32) - jnp.int32(0x4B400000)
    ni = nint.astype(jnp.float32)
    sgn = 1.0 - 2.0 * (nint & 1).astype(jnp.float32)
    r = a - ni * jnp.float32(3.140625)
    r = r - ni * jnp.float32(0.0009676536)
    r = r - ni * jnp.float32(5.126688e-12)
    r2 = r * r
    p = r * (jnp.float32(9.999999970017e-01)
             + r2 * (jnp.float32(-1.666665997157e-01)
                     + r2 * (jnp.float32(8.333097587152e-03)
                             + r2 * (jnp.float32(-1.981248784256e-04)
                                     + r2 * jnp.float32(2.612907779947e-06)))))
    sv = (am * (p * sgn)).astype(jnp.bfloat16)
    fl = lax.broadcasted_iota(jnp.int32, (128, 128), 0)
    fo = lax.broadcasted_iota(jnp.int32, (128, 128), 1)
    fold = jnp.where((fl < S) & (fo == S + fl % T), 1.0, 0.0).astype(jnp.bfloat16)
    temb = lax.dot_general(sv, fold, (((1,), (0,)), ((), ())),
                           preferred_element_type=jnp.float32)
    xb = (g0 + temb).astype(jnp.bfloat16)

    rel = rel_ref[...]
    ri = lax.broadcasted_iota(jnp.int32, (BLK, R2), 1)
    oh = (rel == ri).astype(jnp.float32)
    acc = lax.dot_general(oh, b_ref[...], (((1,), (0,)), ((), ())),
                          preferred_element_type=jnp.float32)
    for r in range(R2):
        zr = lax.dot_general(xb, w_ref[r], (((1,), (0,)), ((), ())),
                             preferred_element_type=jnp.float32)
        acc = acc + oh[:, r:r + 1] * zr
    acc = jnp.maximum(acc, 0.0)

    prow = lax.broadcasted_iota(jnp.int32, (EPB, BLK), 0)
    pcol = lax.broadcasted_iota(jnp.int32, (EPB, BLK), 1)
    pool = jnp.where(pcol // K == prow, 1.0 / K, 0.0)
    i = pl.program_id(0)
    pool_ref[pl.ds(i * EPB, EPB), :] = lax.dot_general(
        pool, acc, (((1,), (0,)), ((), ())),
        preferred_element_type=jnp.float32)

    @pl.when(i == NBLK - 1)
    def _():
        h = pool_ref[0:B, :]
        t = pool_ref[B:2 * B, :]
        ridx = rq_ref[...]
        i16 = lax.broadcasted_iota(jnp.int32, (B, NUM_REL), 1)
        ohq = (ridx == i16).astype(jnp.float32)
        rr = lax.dot_general(ohq, re_ref[...], (((1,), (0,)), ((), ())),
                             preferred_element_type=jnp.float32)
        diff = h + rr - t
        s = jnp.sum(diff * diff, axis=1, keepdims=True)
        out_ref[...] = -jnp.sqrt(s + 1e-12)


_transform = pl.pallas_call(
    _transform_body,
    grid=(NBLK,),
    in_specs=[
        pl.BlockSpec((BLK, TBW), lambda i: (i, 0)),
        pl.BlockSpec((BLK, 4), lambda i: (i, 0)),
        pl.BlockSpec((BLK, 1), lambda i: (i, 0)),
        pl.BlockSpec((R2, D, D), lambda i: (0, 0, 0)),
        pl.BlockSpec((R2, D), lambda i: (0, 0)),
        pl.BlockSpec((B, 1), lambda i: (0, 0)),
        pl.BlockSpec((NUM_REL, D), lambda i: (0, 0)),
    ],
    out_specs=pl.BlockSpec((B, 1), lambda i: (0, 0)),
    out_shape=jax.ShapeDtypeStruct((B, 1), jnp.float32),
    scratch_shapes=[pltpu.VMEM((2 * B, D), jnp.float32)],
)


def kernel(heads, rels, tails, years, months, days, neighbor_idx, neighbor_rel,
           ny, nm, nd, ent_embs, rel_embs,
           y_freq, y_phi, y_amp, m_freq, m_phi, m_amp,
           d_freq, d_phi, d_amp, W, b):
    zpad = jnp.zeros((NUM_ENT, T), jnp.float32)
    ent_g = jnp.concatenate([ent_embs, zpad], axis=1)
    fr_g = jnp.concatenate([y_freq, m_freq, d_freq, zpad], axis=1)
    ph_g = jnp.concatenate([y_phi, m_phi, d_phi, zpad], axis=1)
    am_g = jnp.concatenate([y_amp, m_amp, d_amp, zpad], axis=1)

    def pack2(lo, hi):
        lob = lax.bitcast_convert_type(lo.astype(jnp.bfloat16),
                                       jnp.uint16).astype(jnp.uint32)
        hib = lax.bitcast_convert_type(hi.astype(jnp.bfloat16),
                                       jnp.uint16).astype(jnp.uint32)
        return lax.bitcast_convert_type(lob | (hib << 16), jnp.int32)

    table = jnp.concatenate([pack2(ent_g, fr_g), pack2(ph_g, am_g)], axis=1)
    idx = neighbor_idx.reshape(NTOK).astype(jnp.int32)
    g = _build_sc_gather()(table, idx)
    tv = jnp.concatenate(
        [ny, nm, nd, jnp.zeros_like(ny)], axis=-1).reshape(NTOK, 4)
    rel = neighbor_rel.reshape(NTOK, 1).astype(jnp.int32)
    scores = _transform(g, tv, rel, W.astype(jnp.bfloat16), b,
                        rels.reshape(B, 1).astype(jnp.int32), rel_embs)
    return scores.reshape(B)
